# baseline (speedup 1.0000x reference)
import jax
import jax.numpy as jnp
from jax import lax
from jax.experimental import pallas as pl
from jax.experimental.pallas import tpu as pltpu
from jax.experimental.pallas import tpu_sc as plsc

N_NODES = 10000
N_EDGES = 320000

NC = 2
NS = 16
NW = NC * NS
E_PER_W = 10240
E_PAD = NW * E_PER_W
N_ACC = 10240
ROWS_PER_SUB = N_ACC // NS
PAD_SPREAD = 64
DW = 8

_mesh = plsc.VectorSubcoreMesh(core_axis_name="c", subcore_axis_name="s")
_sc_params = pltpu.CompilerParams(use_tc_tiling_on_sc=False)


def _deg_body(dst_hbm, ones_hbm, zeros_hbm, out_hbm, idx_v, ones_v, acc, sem):
    c = lax.axis_index("c")
    s = lax.axis_index("s")
    wid = s * NC + c
    r0 = s * ROWS_PER_SUB
    pltpu.sync_copy(zeros_hbm.at[pl.ds(r0, ROWS_PER_SUB)],
                    acc.at[pl.ds(r0, ROWS_PER_SUB)])
    pltpu.sync_copy(ones_hbm, ones_v)
    pltpu.sync_copy(dst_hbm.at[wid], idx_v)
    plsc.subcore_barrier()

    @pl.loop(0, 80)
    def _fire(j):
        pltpu.async_copy(ones_v, acc.at[idx_v.at[j]], sem, add=True)

    @pl.loop(0, 80)
    def _drain(j):
        pltpu.make_async_copy(ones_v, acc.at[idx_v.at[0]], sem).wait()

    plsc.subcore_barrier()
    pltpu.sync_copy(acc.at[pl.ds(r0, ROWS_PER_SUB)],
                    out_hbm.at[c, pl.ds(r0, ROWS_PER_SUB)])


_deg_kernel = pl.kernel(
    _deg_body,
    out_type=jax.ShapeDtypeStruct((NC, N_ACC, DW), jnp.float32),
    mesh=_mesh,
    compiler_params=_sc_params,
    scratch_types=[
        pltpu.VMEM((80, 128), jnp.int32),
        pltpu.VMEM((128, DW), jnp.float32),
        pltpu.VMEM_SHARED((N_ACC, DW), jnp.float32),
        pltpu.SemaphoreType.DMA,
    ],
)


def _make_spmm(width, le):
    nl = E_PER_W // le
    assert (nl - 4) % 4 == 0

    def body(src_hbm, dst_hbm, g_hbm, zeros_hbm, out_hbm,
             sidx, didx, r0b, r1b, r2b, r3b, acc,
             g0, g1, g2, g3, s0, s1, s2, s3):
        c = lax.axis_index("c")
        s = lax.axis_index("s")
        wid = s * NC + c
        r0 = s * ROWS_PER_SUB
        rows = (r0b, r1b, r2b, r3b)
        gsem = (g0, g1, g2, g3)
        ssem = (s0, s1, s2, s3)
        pltpu.sync_copy(zeros_hbm.at[pl.ds(r0, ROWS_PER_SUB)],
                        acc.at[pl.ds(r0, ROWS_PER_SUB)])
        pltpu.sync_copy(src_hbm.at[wid], sidx)
        pltpu.sync_copy(dst_hbm.at[wid], didx)
        plsc.subcore_barrier()

        def start_g(j, b):
            pltpu.async_copy(g_hbm.at[sidx.at[j]], rows[b], gsem[b])

        def wait_g(b):
            pltpu.make_async_copy(g_hbm.at[sidx.at[0]], rows[b], gsem[b]).wait()

        def start_s(j, b):
            pltpu.async_copy(rows[b], acc.at[didx.at[j]], ssem[b], add=True)

        def wait_s(b):
            pltpu.make_async_copy(rows[b], acc.at[didx.at[0]], ssem[b]).wait()

        start_g(0, 0)
        start_g(1, 1)
        wait_g(0)
        start_s(0, 0)
        start_g(2, 2)
        wait_g(1)
        start_s(1, 1)
        start_g(3, 3)

        @pl.loop(0, (nl - 4) // 4)
        def _pipe(q):
            for u in range(4):
                j = 2 + 4 * q + u
                b = (2 + u) % 4
                wait_g(b)
                start_s(j, b)
                wait_s((b + 2) % 4)
                start_g(j + 2, (b + 2) % 4)

        for j in (nl - 2, nl - 1):
            b = j % 4
            wait_g(b)
            start_s(j, b)
        for b in range(4):
            wait_s(b)

        plsc.subcore_barrier()
        pltpu.sync_copy(acc.at[pl.ds(r0, ROWS_PER_SUB)],
                        out_hbm.at[c, pl.ds(r0, ROWS_PER_SUB)])

    return pl.kernel(
        body,
        out_type=jax.ShapeDtypeStruct((NC, N_ACC, width), jnp.float32),
        mesh=_mesh,
        compiler_params=_sc_params,
        scratch_types=[
            pltpu.VMEM((nl, le), jnp.int32),
            pltpu.VMEM((nl, le), jnp.int32),
            pltpu.VMEM((le, width), jnp.float32),
            pltpu.VMEM((le, width), jnp.float32),
            pltpu.VMEM((le, width), jnp.float32),
            pltpu.VMEM((le, width), jnp.float32),
            pltpu.VMEM_SHARED((N_ACC, width), jnp.float32),
        ] + [pltpu.SemaphoreType.DMA] * 8,
    )


_spmm64 = _make_spmm(64, 128)
_spmm32 = _make_spmm(32, 128)


def _tc_matmul1(x_pad, W1):
    def body(x_ref, w_ref, o_ref):
        o_ref[...] = jnp.dot(x_ref[...], w_ref[...],
                             preferred_element_type=jnp.float32)
    return pl.pallas_call(
        body,
        out_shape=jax.ShapeDtypeStruct((N_ACC, 64), jnp.float32),
    )(x_pad, W1)


def _dinv_of(dp_val):
    deg = dp_val[0, :, 0:1] + dp_val[1, :, 0:1] + 1.0
    row = lax.broadcasted_iota(jnp.int32, (N_ACC, 1), 0)
    return jnp.where(row < N_NODES, lax.rsqrt(deg), 0.0)


def _tc_scale(h1, dp):
    def body(h_ref, dp_ref, g_ref):
        g_ref[...] = _dinv_of(dp_ref[...]) * h_ref[...]
    return pl.pallas_call(
        body,
        out_shape=jax.ShapeDtypeStruct((N_ACC, 64), jnp.float32),
    )(h1, dp)


def _tc_layer2(p1, g1, dp, b1, W2):
    def body(p_ref, g_ref, dp_ref, b_ref, w_ref, o_ref):
        pv = p_ref[...]
        agg = pv[0] + pv[1] + g_ref[...]
        dinv = _dinv_of(dp_ref[...])
        h = jnp.maximum(dinv * agg + b_ref[...], 0.0)
        o_ref[...] = dinv * jnp.dot(h, w_ref[...],
                                    preferred_element_type=jnp.float32)
    return pl.pallas_call(
        body,
        out_shape=jax.ShapeDtypeStruct((N_ACC, 32), jnp.float32),
    )(p1, g1, dp, b1, W2)


def _tc_head(p2, g2, dp, b2, Wfc, bfc):
    def body(p_ref, g_ref, dp_ref, b_ref, w_ref, bf_ref, o_ref):
        pv = p_ref[...]
        agg = pv[0] + pv[1] + g_ref[...]
        h = jnp.maximum(_dinv_of(dp_ref[...]) * agg + b_ref[...], 0.0)
        o_ref[...] = jnp.dot(h, w_ref[...],
                             preferred_element_type=jnp.float32) + bf_ref[...]
    return pl.pallas_call(
        body,
        out_shape=jax.ShapeDtypeStruct((N_ACC, 1), jnp.float32),
    )(p2, g2, dp, b2, Wfc, bfc)


def kernel(x, edge_index, W1, b1, W2, b2, Wfc, bfc):
    src = edge_index[0].astype(jnp.int32)
    dst = edge_index[1].astype(jnp.int32)
    pad = N_NODES + (jnp.arange(E_PAD - N_EDGES, dtype=jnp.int32) % PAD_SPREAD)
    src_f = jnp.concatenate([src, pad])
    dst_f = jnp.concatenate([dst, pad])
    src3 = src_f.reshape(NW, 80, 128)
    dst3 = dst_f.reshape(NW, 80, 128)
    x_pad = jnp.pad(x, ((0, N_ACC - N_NODES), (0, 0)))

    ones_dw = jnp.ones((128, DW), jnp.float32)
    zeros_dw = jnp.zeros((N_ACC, DW), jnp.float32)
    zeros64 = jnp.zeros((N_ACC, 64), jnp.float32)
    zeros32 = jnp.zeros((N_ACC, 32), jnp.float32)

    dp = _deg_kernel(dst3, ones_dw, zeros_dw)
    h1 = _tc_matmul1(x_pad, W1)
    g1 = _tc_scale(h1, dp)
    p1 = _spmm64(src3, dst3, g1, zeros64)
    g2 = _tc_layer2(p1, g1, dp, b1.reshape(1, 64), W2)
    p2 = _spmm32(src3, dst3, g2, zeros32)
    out = _tc_head(p2, g2, dp, b2.reshape(1, 32), Wfc, bfc.reshape(1, 1))
    return out[:N_NODES]


# speedup vs baseline: 1.0815x; 1.0815x over previous
import jax
import jax.numpy as jnp
from jax import lax
from jax.experimental import pallas as pl
from jax.experimental.pallas import tpu as pltpu
from jax.experimental.pallas import tpu_sc as plsc

N_NODES = 10000
N_EDGES = 320000

NC = 2
NS = 16
NW = NC * NS
E_PER_W = 10240
E_PAD = NW * E_PER_W
N_ACC = 10240
ROWS_PER_SUB = N_ACC // NS
PAD_SPREAD = 64
DW = 8

_mesh = plsc.VectorSubcoreMesh(core_axis_name="c", subcore_axis_name="s")
_sc_params = pltpu.CompilerParams(use_tc_tiling_on_sc=False)


DEG_LE = 1024
DEG_NL = E_PER_W // DEG_LE


def _deg_body(dst_hbm, ones_hbm, zeros_hbm, out_hbm,
              ia, ib, ones_v, acc, sem, ija, ijb):
    c = lax.axis_index("c")
    s = lax.axis_index("s")
    wid = s * NC + c
    r0 = s * ROWS_PER_SUB
    base = wid * E_PER_W
    idx = (ia, ib)
    isem = (ija, ijb)
    pltpu.sync_copy(zeros_hbm.at[pl.ds(r0, ROWS_PER_SUB)],
                    acc.at[pl.ds(r0, ROWS_PER_SUB)])
    pltpu.sync_copy(ones_hbm, ones_v)
    plsc.subcore_barrier()

    def start_i(j, b):
        pltpu.async_copy(dst_hbm.at[pl.ds(base + j * DEG_LE, DEG_LE)],
                         idx[b], isem[b])

    def wait_i(b):
        pltpu.make_async_copy(dst_hbm.at[pl.ds(base, DEG_LE)],
                              idx[b], isem[b]).wait()

    start_i(0, 0)
    start_i(1, 1)

    @pl.loop(0, DEG_NL // 2)
    def _fire(q):
        for u in range(2):
            j = 2 * q + u
            wait_i(u)
            pltpu.async_copy(ones_v, acc.at[idx[u]], sem, add=True)
            pltpu.make_async_copy(ones_v, acc.at[idx[0]], sem).wait()

            @pl.when(j + 2 < DEG_NL)
            def _():
                start_i(j + 2, u)

    plsc.subcore_barrier()
    pltpu.sync_copy(acc.at[pl.ds(r0, ROWS_PER_SUB)],
                    out_hbm.at[c, pl.ds(r0, ROWS_PER_SUB)])


_deg_kernel = pl.kernel(
    _deg_body,
    out_type=jax.ShapeDtypeStruct((NC, N_ACC, DW), jnp.float32),
    mesh=_mesh,
    compiler_params=_sc_params,
    scratch_types=[
        pltpu.VMEM((DEG_LE,), jnp.int32),
        pltpu.VMEM((DEG_LE,), jnp.int32),
        pltpu.VMEM((DEG_LE, DW), jnp.float32),
        pltpu.VMEM_SHARED((N_ACC, DW), jnp.float32),
        pltpu.SemaphoreType.DMA,
        pltpu.SemaphoreType.DMA,
        pltpu.SemaphoreType.DMA,
    ],
)


def _make_spmm(width, le):
    nl = E_PER_W // le
    assert (nl - 4) % 4 == 0

    def body(src_hbm, dst_hbm, g_hbm, zeros_hbm, out_hbm,
             si0, si1, si2, si3, di0, di1, di2, di3,
             r0b, r1b, r2b, r3b, acc,
             g0, g1, g2, g3, s0, s1, s2, s3, i0, i1, i2, i3):
        c = lax.axis_index("c")
        s = lax.axis_index("s")
        wid = s * NC + c
        r0 = s * ROWS_PER_SUB
        base = wid * E_PER_W
        sidx = (si0, si1, si2, si3)
        didx = (di0, di1, di2, di3)
        rows = (r0b, r1b, r2b, r3b)
        gsem = (g0, g1, g2, g3)
        ssem = (s0, s1, s2, s3)
        isem = (i0, i1, i2, i3)
        pltpu.sync_copy(zeros_hbm.at[pl.ds(r0, ROWS_PER_SUB)],
                        acc.at[pl.ds(r0, ROWS_PER_SUB)])
        plsc.subcore_barrier()

        def start_i(j, b):
            pltpu.async_copy(src_hbm.at[pl.ds(base + j * le, le)],
                             sidx[b], isem[b])
            pltpu.async_copy(dst_hbm.at[pl.ds(base + j * le, le)],
                             didx[b], isem[b])

        def wait_i(b):
            pltpu.make_async_copy(src_hbm.at[pl.ds(base, le)],
                                  sidx[b], isem[b]).wait()
            pltpu.make_async_copy(dst_hbm.at[pl.ds(base, le)],
                                  didx[b], isem[b]).wait()

        def start_g(b):
            pltpu.async_copy(g_hbm.at[sidx[b]], rows[b], gsem[b])

        def wait_g(b):
            pltpu.make_async_copy(g_hbm.at[sidx[0]], rows[b], gsem[b]).wait()

        def start_s(b):
            pltpu.async_copy(rows[b], acc.at[didx[b]], ssem[b], add=True)

        def wait_s(b):
            pltpu.make_async_copy(rows[b], acc.at[didx[0]], ssem[b]).wait()

        for b in range(4):
            start_i(b, b)
        wait_i(0)
        start_g(0)
        wait_i(1)
        start_g(1)
        wait_g(0)
        start_s(0)
        wait_i(2)
        start_g(2)
        wait_g(1)
        start_s(1)
        wait_i(3)
        start_g(3)

        @pl.loop(0, (nl - 4) // 4)
        def _pipe(q):
            for u in range(4):
                j = 2 + 4 * q + u
                b = (2 + u) % 4
                bn = (b + 2) % 4
                wait_g(b)
                start_s(b)
                wait_s(bn)
                start_i(j + 2, bn)
                wait_i(bn)
                start_g(bn)

        for j in (nl - 2, nl - 1):
            b = j % 4
            wait_g(b)
            start_s(b)
        for b in range(4):
            wait_s(b)

        plsc.subcore_barrier()
        pltpu.sync_copy(acc.at[pl.ds(r0, ROWS_PER_SUB)],
                        out_hbm.at[c, pl.ds(r0, ROWS_PER_SUB)])

    return pl.kernel(
        body,
        out_type=jax.ShapeDtypeStruct((NC, N_ACC, width), jnp.float32),
        mesh=_mesh,
        compiler_params=_sc_params,
        scratch_types=[
            pltpu.VMEM((le,), jnp.int32),
            pltpu.VMEM((le,), jnp.int32),
            pltpu.VMEM((le,), jnp.int32),
            pltpu.VMEM((le,), jnp.int32),
            pltpu.VMEM((le,), jnp.int32),
            pltpu.VMEM((le,), jnp.int32),
            pltpu.VMEM((le,), jnp.int32),
            pltpu.VMEM((le,), jnp.int32),
            pltpu.VMEM((le, width), jnp.float32),
            pltpu.VMEM((le, width), jnp.float32),
            pltpu.VMEM((le, width), jnp.float32),
            pltpu.VMEM((le, width), jnp.float32),
            pltpu.VMEM_SHARED((N_ACC, width), jnp.float32),
        ] + [pltpu.SemaphoreType.DMA] * 12,
    )


_spmm64 = _make_spmm(64, 256)
_spmm32 = _make_spmm(32, 512)


def _tc_matmul1(x, W1):
    def body(x_ref, w_ref, o_ref):
        o_ref[0:N_NODES, :] = jnp.dot(x_ref[...], w_ref[...],
                                      preferred_element_type=jnp.float32)
        o_ref[N_NODES:N_ACC, :] = jnp.zeros((N_ACC - N_NODES, 64), jnp.float32)
    return pl.pallas_call(
        body,
        out_shape=jax.ShapeDtypeStruct((N_ACC, 64), jnp.float32),
    )(x, W1)


def _dinv_of(dp_val):
    deg = dp_val[0, :, 0:1] + dp_val[1, :, 0:1] + 1.0
    row = lax.broadcasted_iota(jnp.int32, (N_ACC, 1), 0)
    return jnp.where(row < N_NODES, lax.rsqrt(deg), 0.0)


def _tc_scale(h1, dp):
    def body(h_ref, dp_ref, g_ref):
        g_ref[...] = _dinv_of(dp_ref[...]) * h_ref[...]
    return pl.pallas_call(
        body,
        out_shape=jax.ShapeDtypeStruct((N_ACC, 64), jnp.float32),
    )(h1, dp)


def _tc_layer2(p1, g1, dp, b1, W2):
    def body(p_ref, g_ref, dp_ref, b_ref, w_ref, o_ref):
        pv = p_ref[...]
        agg = pv[0] + pv[1] + g_ref[...]
        dinv = _dinv_of(dp_ref[...])
        h = jnp.maximum(dinv * agg + b_ref[...], 0.0)
        o_ref[...] = dinv * jnp.dot(h, w_ref[...],
                                    preferred_element_type=jnp.float32)
    return pl.pallas_call(
        body,
        out_shape=jax.ShapeDtypeStruct((N_ACC, 32), jnp.float32),
    )(p1, g1, dp, b1, W2)


def _tc_head(p2, g2, dp, b2, Wfc, bfc):
    def body(p_ref, g_ref, dp_ref, b_ref, w_ref, bf_ref, o_ref):
        pv = p_ref[...]
        agg = pv[0] + pv[1] + g_ref[...]
        h = jnp.maximum(_dinv_of(dp_ref[...]) * agg + b_ref[...], 0.0)
        o_ref[...] = jnp.dot(h, w_ref[...],
                             preferred_element_type=jnp.float32) + bf_ref[...]
    return pl.pallas_call(
        body,
        out_shape=jax.ShapeDtypeStruct((N_ACC, 1), jnp.float32),
    )(p2, g2, dp, b2, Wfc, bfc)


def kernel(x, edge_index, W1, b1, W2, b2, Wfc, bfc):
    src = edge_index[0].astype(jnp.int32)
    dst = edge_index[1].astype(jnp.int32)
    pad = N_NODES + (jnp.arange(E_PAD - N_EDGES, dtype=jnp.int32) % PAD_SPREAD)
    src_f = jnp.concatenate([src, pad])
    dst_f = jnp.concatenate([dst, pad])

    ones_dw = jnp.ones((DEG_LE, DW), jnp.float32)
    zeros_dw = jnp.zeros((N_ACC, DW), jnp.float32)
    zeros64 = jnp.zeros((N_ACC, 64), jnp.float32)
    zeros32 = jnp.zeros((N_ACC, 32), jnp.float32)

    dp = _deg_kernel(dst_f, ones_dw, zeros_dw)
    h1 = _tc_matmul1(x, W1)
    g1 = _tc_scale(h1, dp)
    p1 = _spmm64(src_f, dst_f, g1, zeros64)
    g2 = _tc_layer2(p1, g1, dp, b1.reshape(1, 64), W2)
    p2 = _spmm32(src_f, dst_f, g2, zeros32)
    out = _tc_head(p2, g2, dp, b2.reshape(1, 32), Wfc, bfc.reshape(1, 1))
    return out[:N_NODES]
